# baseline (speedup 1.0000x reference)
import dataclasses
import functools

import jax
import jax.numpy as jnp
from jax import lax
from jax.experimental import pallas as pl
from jax.experimental.pallas import tpu as pltpu
from jax.experimental.pallas import tpu_sc as plsc

N_NODES = 10000
N_EDGES = 320000
D = 128
L = 16

NC = 2
NS = 16
NW = NC * NS
K = 32
NB = 4
NCHUNK = N_EDGES // K
CPW = NCHUNK // NW
XTRA = NCHUNK - CPW * NW
NP = 10240
RPS = NP // NS
DR = NP // D

_mesh = plsc.VectorSubcoreMesh(core_axis_name="c", subcore_axis_name="s")

_cp = pltpu.CompilerParams()
if "needs_layout_passes" in pltpu.CompilerParams.__dataclass_fields__:
    _cp = dataclasses.replace(_cp, needs_layout_passes=False)


@functools.partial(
    pl.kernel,
    out_type=(
        jax.ShapeDtypeStruct((NC, NP, D), jnp.float32),
        jax.ShapeDtypeStruct((NC, NS, DR, D), jnp.float32),
    ),
    mesh=_mesh,
    scratch_types=[
        [pltpu.VMEM((2, K), jnp.int32)] * NB,
        [pltpu.VMEM((K, D), jnp.float32)] * NB,
        pltpu.VMEM((DR, D), jnp.float32),
        pltpu.VMEM_SHARED((NP, D), jnp.float32),
        [pltpu.SemaphoreType.DMA] * NB,
        [pltpu.SemaphoreType.DMA] * NB,
        [pltpu.SemaphoreType.DMA] * NB,
    ],
    compiler_params=_cp,
)
def _sc_aggregate(x_hbm, sidi_hbm, zh_hbm, zd_hbm,
                  ph_hbm, pd_hbm,
                  sidi, rows, dl_v, acc_h, semi, semg, sems):
    c = lax.axis_index("c")
    s = lax.axis_index("s")
    wid = c * NS + s
    r0 = s * RPS
    cbase = wid * CPW

    pltpu.sync_copy(zh_hbm, acc_h.at[pl.ds(r0, RPS)])
    pltpu.sync_copy(zd_hbm, dl_v)
    plsc.subcore_barrier()

    ones16 = jnp.full((L,), 1.0, jnp.float32)

    def start_idx(g, b):
        pltpu.async_copy(sidi_hbm.at[g], sidi[b], semi[b])

    def wait_idx(b):
        pltpu.make_async_copy(sidi_hbm.at[0], sidi[b], semi[b]).wait()

    def start_gather(b):
        pltpu.async_copy(x_hbm.at[sidi[b].at[0]], rows[b], semg[b])

    def wait_gather(b):
        pltpu.make_async_copy(x_hbm.at[sidi[b].at[0]], rows[b],
                              semg[b]).wait()

    def start_scatter(b):
        pltpu.async_copy(rows[b], acc_h.at[sidi[b].at[1]], sems[b], add=True)

    def drain_scatter(b):
        pltpu.make_async_copy(rows[b], acc_h.at[sidi[b].at[1]],
                              sems[b]).wait()

    def deg(b):
        for t in range(K // L):
            idx16 = sidi[b][1, pl.ds(t * L, L)]
            plsc.addupdate_scatter(
                dl_v, [lax.shift_right_logical(idx16, 7),
                       lax.bitwise_and(idx16, 127)], ones16)

    start_idx(cbase, 0)
    start_idx(cbase + 1, 1)
    start_idx(cbase + 2, 2)
    wait_idx(0)
    start_gather(0)
    wait_idx(1)
    start_gather(1)

    @pl.loop(0, CPW // NB)
    def _(jj):
        t0 = jj * NB
        for b in range(NB):
            t = t0 + b

            @pl.when(t >= 1)
            def _():
                drain_scatter((b + NB - 1) % NB)

            @pl.when(t + 3 < CPW)
            def _():
                start_idx(cbase + t + 3, (b + 3) % NB)

            @pl.when(t + 2 < CPW)
            def _():
                wait_idx((b + 2) % NB)
                start_gather((b + 2) % NB)

            wait_gather(b)
            start_scatter(b)
            deg(b)

    drain_scatter((CPW - 1) % NB)

    @pl.when(wid < XTRA)
    def _():
        g = NW * CPW + wid
        start_idx(g, 0)
        wait_idx(0)
        start_gather(0)
        wait_gather(0)
        pltpu.sync_copy(rows[0], acc_h.at[sidi[0].at[1]], add=True)
        deg(0)

    plsc.subcore_barrier()

    pltpu.sync_copy(acc_h.at[pl.ds(r0, RPS)], ph_hbm.at[c].at[pl.ds(r0, RPS)])
    pltpu.sync_copy(dl_v, pd_hbm.at[c].at[s])


_RB = 1024
_SB = _RB // D


def _tc_body(ph_ref, pd_ref, wt_ref, b_ref, o_ref):
    h = ph_ref[0] + ph_ref[1]
    dall = jnp.sum(pd_ref[...], axis=(0, 1))
    ri = lax.broadcasted_iota(jnp.int32, (D, D), 0)
    ci = lax.broadcasted_iota(jnp.int32, (D, D), 1)
    eye = ri == ci
    wt = wt_ref[...]
    bb = b_ref[...]
    for r in range(_SB):
        recip = 1.0 / jnp.maximum(dall[r], 1.0)
        diag = jnp.where(eye, recip, 0.0)
        hr = h[r * D:(r + 1) * D]
        hn = jnp.dot(diag, hr, preferred_element_type=jnp.float32)
        o_ref[pl.ds(r * D, D), :] = (
            jnp.dot(hn, wt, preferred_element_type=jnp.float32) + bb)


def _tc_linear(ph, pd5, wt, b2):
    return pl.pallas_call(
        _tc_body,
        grid=(pl.cdiv(N_NODES, _RB),),
        in_specs=[
            pl.BlockSpec((NC, _RB, D), lambda i: (0, i, 0)),
            pl.BlockSpec((NC, NS, _SB, 1, D), lambda i: (0, 0, i, 0, 0)),
            pl.BlockSpec((D, D), lambda i: (0, 0)),
            pl.BlockSpec((1, D), lambda i: (0, 0)),
        ],
        out_specs=pl.BlockSpec((_RB, D), lambda i: (i, 0)),
        out_shape=jax.ShapeDtypeStruct((N_NODES, D), jnp.float32),
    )(ph, pd5, wt, b2)


def kernel(inputs, edge_index, W, b):
    ei = edge_index.astype(jnp.int32)
    sidi = ei.reshape(2, NCHUNK, K).transpose(1, 0, 2)
    zh = jnp.zeros((RPS, D), jnp.float32)
    zd = jnp.zeros((DR, D), jnp.float32)
    ph, pd = _sc_aggregate(inputs, sidi, zh, zd)
    pd5 = pd.reshape(NC, NS, DR, 1, D)
    return _tc_linear(ph, pd5, W.T, b.reshape(1, D))


# speedup vs baseline: 1.4758x; 1.4758x over previous
import dataclasses
import functools

import jax
import jax.numpy as jnp
from jax import lax
from jax.experimental import pallas as pl
from jax.experimental.pallas import tpu as pltpu
from jax.experimental.pallas import tpu_sc as plsc

N_NODES = 10000
N_EDGES = 320000
D = 128
L = 16

NC = 2
NS = 16
NW = NC * NS
K = 64
NBI = 6
NBR = 4
UNR = 12
NCHUNK = N_EDGES // K
CPW = NCHUNK // NW
XTRA = NCHUNK - CPW * NW
NP = 10240
RPS = NP // NS
DR = NP // D

_mesh = plsc.VectorSubcoreMesh(core_axis_name="c", subcore_axis_name="s")

_cp = pltpu.CompilerParams()
if "needs_layout_passes" in pltpu.CompilerParams.__dataclass_fields__:
    _cp = dataclasses.replace(_cp, needs_layout_passes=False)


@functools.partial(
    pl.kernel,
    out_type=(
        jax.ShapeDtypeStruct((NC, NP, D), jnp.float32),
        jax.ShapeDtypeStruct((NC, NS, DR, D), jnp.float32),
    ),
    mesh=_mesh,
    scratch_types=[
        [pltpu.VMEM((2, K), jnp.int32)] * NBI,
        [pltpu.VMEM((K, D), jnp.float32)] * NBR,
        pltpu.VMEM((DR, D), jnp.float32),
        pltpu.VMEM_SHARED((NP, D), jnp.float32),
        [pltpu.SemaphoreType.DMA] * NBI,
        [pltpu.SemaphoreType.DMA] * NBR,
        [pltpu.SemaphoreType.DMA] * NBR,
    ],
    compiler_params=_cp,
)
def _sc_aggregate(x_hbm, sidi_hbm, zh_hbm, zd_hbm,
                  ph_hbm, pd_hbm,
                  sidi, rows, dl_v, acc_h, semi, semg, sems):
    c = lax.axis_index("c")
    s = lax.axis_index("s")
    wid = c * NS + s
    r0 = s * RPS
    cbase = wid * CPW

    pltpu.sync_copy(zh_hbm, acc_h.at[pl.ds(r0, RPS)])
    pltpu.sync_copy(zd_hbm, dl_v)
    plsc.subcore_barrier()

    ones16 = jnp.full((L,), 1.0, jnp.float32)

    def start_idx(g, bi):
        pltpu.async_copy(sidi_hbm.at[g], sidi[bi], semi[bi])

    def wait_idx(bi):
        pltpu.make_async_copy(sidi_hbm.at[0], sidi[bi], semi[bi]).wait()

    def start_gather(bi, br):
        pltpu.async_copy(x_hbm.at[sidi[bi].at[0]], rows[br], semg[br])

    def wait_gather(bi, br):
        pltpu.make_async_copy(x_hbm.at[sidi[bi].at[0]], rows[br],
                              semg[br]).wait()

    def start_scatter(bi, br):
        pltpu.async_copy(rows[br], acc_h.at[sidi[bi].at[1]], sems[br],
                         add=True)

    def drain_scatter(bi, br):
        pltpu.make_async_copy(rows[br], acc_h.at[sidi[bi].at[1]],
                              sems[br]).wait()

    def deg(bi):
        for t in range(K // L):
            idx16 = sidi[bi][1, pl.ds(t * L, L)]
            plsc.addupdate_scatter(
                dl_v, [lax.shift_right_logical(idx16, 7),
                       lax.bitwise_and(idx16, 127)], ones16)

    for i in range(5):
        start_idx(cbase + i, i)
    for i in range(3):
        wait_idx(i)
        start_gather(i, i)

    @pl.loop(0, CPW // UNR)
    def _(jj):
        t0 = jj * UNR
        for u in range(UNR):
            t = t0 + u

            @pl.when(t >= 1)
            def _():
                drain_scatter((u + NBI - 1) % NBI, (u + NBR - 1) % NBR)

            @pl.when(t + 5 < CPW)
            def _():
                start_idx(cbase + t + 5, (u + 5) % NBI)

            @pl.when(t + 3 < CPW)
            def _():
                wait_idx((u + 3) % NBI)
                start_gather((u + 3) % NBI, (u + 3) % NBR)

            wait_gather(u % NBI, u % NBR)
            start_scatter(u % NBI, u % NBR)
            deg(u % NBI)

    drain_scatter((CPW - 1) % NBI, (CPW - 1) % NBR)

    @pl.when(wid < XTRA)
    def _():
        g = NW * CPW + wid
        start_idx(g, 0)
        wait_idx(0)
        start_gather(0, 0)
        wait_gather(0, 0)
        pltpu.sync_copy(rows[0], acc_h.at[sidi[0].at[1]], add=True)
        deg(0)

    plsc.subcore_barrier()

    pltpu.sync_copy(acc_h.at[pl.ds(r0, RPS)], ph_hbm.at[c].at[pl.ds(r0, RPS)])
    pltpu.sync_copy(dl_v, pd_hbm.at[c].at[s])


_RB = 1024
_SB = _RB // D


def _tc_body(ph_ref, pd_ref, wt_ref, b_ref, o_ref):
    h = ph_ref[0] + ph_ref[1]
    dall = jnp.sum(pd_ref[...], axis=(0, 1))
    ri = lax.broadcasted_iota(jnp.int32, (D, D), 0)
    ci = lax.broadcasted_iota(jnp.int32, (D, D), 1)
    eye = ri == ci
    wt = wt_ref[...]
    bb = b_ref[...]
    for r in range(_SB):
        recip = 1.0 / jnp.maximum(dall[r], 1.0)
        diag = jnp.where(eye, recip, 0.0)
        hr = h[r * D:(r + 1) * D]
        hn = jnp.dot(diag, hr, preferred_element_type=jnp.float32)
        o_ref[pl.ds(r * D, D), :] = (
            jnp.dot(hn, wt, preferred_element_type=jnp.float32) + bb)


def _tc_linear(ph, pd5, wt, b2):
    return pl.pallas_call(
        _tc_body,
        grid=(pl.cdiv(N_NODES, _RB),),
        in_specs=[
            pl.BlockSpec((NC, _RB, D), lambda i: (0, i, 0)),
            pl.BlockSpec((NC, NS, _SB, 1, D), lambda i: (0, 0, i, 0, 0)),
            pl.BlockSpec((D, D), lambda i: (0, 0)),
            pl.BlockSpec((1, D), lambda i: (0, 0)),
        ],
        out_specs=pl.BlockSpec((_RB, D), lambda i: (i, 0)),
        out_shape=jax.ShapeDtypeStruct((N_NODES, D), jnp.float32),
    )(ph, pd5, wt, b2)


def kernel(inputs, edge_index, W, b):
    ei = edge_index.astype(jnp.int32)
    sidi = ei.reshape(2, NCHUNK, K).transpose(1, 0, 2)
    zh = jnp.zeros((RPS, D), jnp.float32)
    zd = jnp.zeros((DR, D), jnp.float32)
    ph, pd = _sc_aggregate(inputs, sidi, zh, zd)
    pd5 = pd.reshape(NC, NS, DR, 1, D)
    return _tc_linear(ph, pd5, W.T, b.reshape(1, D))


# speedup vs baseline: 1.5019x; 1.0176x over previous
import dataclasses
import functools

import jax
import jax.numpy as jnp
from jax import lax
from jax.experimental import pallas as pl
from jax.experimental.pallas import tpu as pltpu
from jax.experimental.pallas import tpu_sc as plsc

N_NODES = 10000
N_EDGES = 320000
D = 128
L = 16

NC = 2
NS = 16
NW = NC * NS
K = 64
NBI = 6
NBR = 4
UNR = 12
NCHUNK = N_EDGES // K
CPW = NCHUNK // NW
XTRA = NCHUNK - CPW * NW
NP = 10240
RPS = NP // NS
DR = NP // D

_mesh = plsc.VectorSubcoreMesh(core_axis_name="c", subcore_axis_name="s")

_cp = pltpu.CompilerParams()
if "needs_layout_passes" in pltpu.CompilerParams.__dataclass_fields__:
    _cp = dataclasses.replace(_cp, needs_layout_passes=False)


@functools.partial(
    pl.kernel,
    out_type=(
        jax.ShapeDtypeStruct((NC, NP, D), jnp.float32),
        jax.ShapeDtypeStruct((NC, NS, DR, D), jnp.float32),
    ),
    mesh=_mesh,
    scratch_types=[
        [pltpu.VMEM((2, K), jnp.int32)] * NBI,
        [pltpu.VMEM((K, D), jnp.float32)] * NBR,
        pltpu.VMEM((DR, D), jnp.float32),
        pltpu.VMEM_SHARED((NP, D), jnp.float32),
        [pltpu.SemaphoreType.DMA] * NBI,
        [pltpu.SemaphoreType.DMA] * NBR,
        [pltpu.SemaphoreType.DMA] * NBR,
        [pltpu.SemaphoreType.DMA] * 2,
    ],
    compiler_params=_cp,
)
def _sc_aggregate(x_hbm, sidi_hbm, zh_hbm, zd_hbm,
                  ph_hbm, pd_hbm,
                  sidi, rows, dl_v, acc_h, semi, semg, sems, semz):
    c = lax.axis_index("c")
    s = lax.axis_index("s")
    wid = c * NS + s
    r0 = s * RPS
    cbase = wid * CPW

    pltpu.async_copy(zh_hbm, acc_h.at[pl.ds(r0, RPS)], semz[0])
    pltpu.async_copy(zd_hbm, dl_v, semz[1])

    ones16 = jnp.full((L,), 1.0, jnp.float32)

    def start_idx(g, bi):
        pltpu.async_copy(sidi_hbm.at[g], sidi[bi], semi[bi])

    def wait_idx(bi):
        pltpu.make_async_copy(sidi_hbm.at[0], sidi[bi], semi[bi]).wait()

    def start_gather(bi, br):
        pltpu.async_copy(x_hbm.at[sidi[bi].at[0]], rows[br], semg[br])

    def wait_gather(bi, br):
        pltpu.make_async_copy(x_hbm.at[sidi[bi].at[0]], rows[br],
                              semg[br]).wait()

    def start_scatter(bi, br):
        pltpu.async_copy(rows[br], acc_h.at[sidi[bi].at[1]], sems[br],
                         add=True)

    def drain_scatter(bi, br):
        pltpu.make_async_copy(rows[br], acc_h.at[sidi[bi].at[1]],
                              sems[br]).wait()

    def deg(bi):
        for t in range(K // L):
            idx16 = sidi[bi][1, pl.ds(t * L, L)]
            plsc.addupdate_scatter(
                dl_v, [lax.shift_right_logical(idx16, 7),
                       lax.bitwise_and(idx16, 127)], ones16)

    for i in range(5):
        start_idx(cbase + i, i)
    for i in range(3):
        wait_idx(i)
        start_gather(i, i)

    pltpu.make_async_copy(zh_hbm, acc_h.at[pl.ds(r0, RPS)], semz[0]).wait()
    pltpu.make_async_copy(zd_hbm, dl_v, semz[1]).wait()
    plsc.subcore_barrier()

    @pl.loop(0, CPW // UNR)
    def _(jj):
        t0 = jj * UNR
        for u in range(UNR):
            t = t0 + u

            @pl.when(t >= 1)
            def _():
                drain_scatter((u + NBI - 1) % NBI, (u + NBR - 1) % NBR)

            @pl.when(t + 5 < CPW)
            def _():
                start_idx(cbase + t + 5, (u + 5) % NBI)

            @pl.when(t + 3 < CPW)
            def _():
                wait_idx((u + 3) % NBI)
                start_gather((u + 3) % NBI, (u + 3) % NBR)

            wait_gather(u % NBI, u % NBR)
            start_scatter(u % NBI, u % NBR)
            deg(u % NBI)

    drain_scatter((CPW - 1) % NBI, (CPW - 1) % NBR)

    @pl.when(wid < XTRA)
    def _():
        g = NW * CPW + wid
        start_idx(g, 0)
        wait_idx(0)
        start_gather(0, 0)
        wait_gather(0, 0)
        pltpu.sync_copy(rows[0], acc_h.at[sidi[0].at[1]], add=True)
        deg(0)

    plsc.subcore_barrier()

    pltpu.sync_copy(acc_h.at[pl.ds(r0, RPS)], ph_hbm.at[c].at[pl.ds(r0, RPS)])
    pltpu.sync_copy(dl_v, pd_hbm.at[c].at[s])


_RB = 1024
_SB = _RB // D


def _tc_body(ph_ref, pd_ref, wt_ref, b_ref, o_ref):
    h = ph_ref[0] + ph_ref[1]
    dall = jnp.sum(pd_ref[...], axis=(0, 1))
    ri = lax.broadcasted_iota(jnp.int32, (D, D), 0)
    ci = lax.broadcasted_iota(jnp.int32, (D, D), 1)
    eye = ri == ci
    wt = wt_ref[...]
    bb = b_ref[...]
    for r in range(_SB):
        recip = 1.0 / jnp.maximum(dall[r], 1.0)
        diag = jnp.where(eye, recip, 0.0)
        hr = h[r * D:(r + 1) * D]
        hn = jnp.dot(diag, hr, preferred_element_type=jnp.float32)
        o_ref[pl.ds(r * D, D), :] = (
            jnp.dot(hn, wt, preferred_element_type=jnp.float32) + bb)


def _tc_linear(ph, pd5, wt, b2):
    return pl.pallas_call(
        _tc_body,
        grid=(pl.cdiv(N_NODES, _RB),),
        in_specs=[
            pl.BlockSpec((NC, _RB, D), lambda i: (0, i, 0)),
            pl.BlockSpec((NC, NS, _SB, 1, D), lambda i: (0, 0, i, 0, 0)),
            pl.BlockSpec((D, D), lambda i: (0, 0)),
            pl.BlockSpec((1, D), lambda i: (0, 0)),
        ],
        out_specs=pl.BlockSpec((_RB, D), lambda i: (i, 0)),
        out_shape=jax.ShapeDtypeStruct((N_NODES, D), jnp.float32),
    )(ph, pd5, wt, b2)


def kernel(inputs, edge_index, W, b):
    ei = edge_index.astype(jnp.int32)
    sidi = ei.reshape(2, NCHUNK, K).transpose(1, 0, 2)
    zh = jnp.zeros((RPS, D), jnp.float32)
    zd = jnp.zeros((DR, D), jnp.float32)
    ph, pd = _sc_aggregate(inputs, sidi, zh, zd)
    pd5 = pd.reshape(NC, NS, DR, 1, D)
    return _tc_linear(ph, pd5, W.T, b.reshape(1, D))


# speedup vs baseline: 1.6649x; 1.1085x over previous
import dataclasses
import functools

import jax
import jax.numpy as jnp
from jax import lax
from jax.experimental import pallas as pl
from jax.experimental.pallas import tpu as pltpu
from jax.experimental.pallas import tpu_sc as plsc

N_NODES = 10000
N_EDGES = 320000
D = 128
L = 16

NC = 2
NS = 16
NW = NC * NS
K = 64
NBI = 6
NBR = 4
UNR = 12
NCHUNK = N_EDGES // K
CPW = NCHUNK // NW
XTRA = NCHUNK - CPW * NW
NP = 10240
RPS = NP // NS
DR = NP // D

_mesh = plsc.VectorSubcoreMesh(core_axis_name="c", subcore_axis_name="s")

_cp = pltpu.CompilerParams()
if "needs_layout_passes" in pltpu.CompilerParams.__dataclass_fields__:
    _cp = dataclasses.replace(_cp, needs_layout_passes=False)


@functools.partial(
    pl.kernel,
    out_type=(
        jax.ShapeDtypeStruct((NC, NP, D), jnp.float32),
        jax.ShapeDtypeStruct((NC, NS, DR, D), jnp.float32),
    ),
    mesh=_mesh,
    scratch_types=[
        [pltpu.VMEM((2, K), jnp.int32)] * NBI,
        [pltpu.VMEM((K, D), jnp.float32)] * NBR,
        pltpu.VMEM((DR, D), jnp.float32),
        pltpu.VMEM_SHARED((NP, D), jnp.float32),
        [pltpu.SemaphoreType.DMA] * NBI,
        [pltpu.SemaphoreType.DMA] * NBR,
        [pltpu.SemaphoreType.DMA] * NBR,
        [pltpu.SemaphoreType.DMA] * 2,
    ],
    compiler_params=_cp,
)
def _sc_aggregate(x_hbm, sidi_hbm, zh_hbm, zd_hbm,
                  ph_hbm, pd_hbm,
                  sidi, rows, dl_v, acc_h, semi, semg, sems, semz):
    c = lax.axis_index("c")
    s = lax.axis_index("s")
    wid = c * NS + s
    r0 = s * RPS
    cbase = wid * CPW

    pltpu.async_copy(zh_hbm, acc_h.at[pl.ds(r0, RPS)], semz[0])
    pltpu.async_copy(zd_hbm, dl_v, semz[1])

    ones16 = jnp.full((L,), 1.0, jnp.float32)

    def start_idx(g, bi):
        pltpu.async_copy(sidi_hbm.at[:, g], sidi[bi], semi[bi])

    def wait_idx(bi):
        pltpu.make_async_copy(sidi_hbm.at[:, 0], sidi[bi], semi[bi]).wait()

    def start_gather(bi, br):
        pltpu.async_copy(x_hbm.at[sidi[bi].at[0]], rows[br], semg[br])

    def wait_gather(bi, br):
        pltpu.make_async_copy(x_hbm.at[sidi[bi].at[0]], rows[br],
                              semg[br]).wait()

    def start_scatter(bi, br):
        pltpu.async_copy(rows[br], acc_h.at[sidi[bi].at[1]], sems[br],
                         add=True)

    def drain_scatter(bi, br):
        pltpu.make_async_copy(rows[br], acc_h.at[sidi[bi].at[1]],
                              sems[br]).wait()

    def deg(bi):
        for t in range(K // L):
            idx16 = sidi[bi][1, pl.ds(t * L, L)]
            plsc.addupdate_scatter(
                dl_v, [lax.shift_right_logical(idx16, 7),
                       lax.bitwise_and(idx16, 127)], ones16)

    for i in range(5):
        start_idx(cbase + i, i)
    for i in range(3):
        wait_idx(i)
        start_gather(i, i)

    pltpu.make_async_copy(zh_hbm, acc_h.at[pl.ds(r0, RPS)], semz[0]).wait()
    pltpu.make_async_copy(zd_hbm, dl_v, semz[1]).wait()
    plsc.subcore_barrier()

    @pl.loop(0, CPW // UNR)
    def _(jj):
        t0 = jj * UNR
        for u in range(UNR):
            t = t0 + u

            @pl.when(t >= 1)
            def _():
                drain_scatter((u + NBI - 1) % NBI, (u + NBR - 1) % NBR)

            @pl.when(t + 5 < CPW)
            def _():
                start_idx(cbase + t + 5, (u + 5) % NBI)

            @pl.when(t + 3 < CPW)
            def _():
                wait_idx((u + 3) % NBI)
                start_gather((u + 3) % NBI, (u + 3) % NBR)

            wait_gather(u % NBI, u % NBR)
            start_scatter(u % NBI, u % NBR)
            deg(u % NBI)

    drain_scatter((CPW - 1) % NBI, (CPW - 1) % NBR)

    @pl.when(wid < XTRA)
    def _():
        g = NW * CPW + wid
        start_idx(g, 0)
        wait_idx(0)
        start_gather(0, 0)
        wait_gather(0, 0)
        pltpu.sync_copy(rows[0], acc_h.at[sidi[0].at[1]], add=True)
        deg(0)

    plsc.subcore_barrier()

    pltpu.sync_copy(acc_h.at[pl.ds(r0, RPS)], ph_hbm.at[c].at[pl.ds(r0, RPS)])
    pltpu.sync_copy(dl_v, pd_hbm.at[c].at[s])


_RB = 1024
_SB = _RB // D


def _tc_body(ph_ref, pd_ref, wt_ref, b_ref, o_ref):
    h = ph_ref[0] + ph_ref[1]
    dall = jnp.sum(pd_ref[...], axis=(0, 1))
    ri = lax.broadcasted_iota(jnp.int32, (D, D), 0)
    ci = lax.broadcasted_iota(jnp.int32, (D, D), 1)
    eye = ri == ci
    wt = wt_ref[...]
    bb = b_ref[...]
    for r in range(_SB):
        recip = 1.0 / jnp.maximum(dall[r], 1.0)
        diag = jnp.where(eye, recip, 0.0)
        hr = h[r * D:(r + 1) * D]
        hn = jnp.dot(diag, hr, preferred_element_type=jnp.float32)
        o_ref[pl.ds(r * D, D), :] = (
            jnp.dot(hn, wt, preferred_element_type=jnp.float32) + bb)


def _tc_linear(ph, pd5, wt, b2):
    return pl.pallas_call(
        _tc_body,
        grid=(pl.cdiv(N_NODES, _RB),),
        in_specs=[
            pl.BlockSpec((NC, _RB, D), lambda i: (0, i, 0)),
            pl.BlockSpec((NC, NS, _SB, 1, D), lambda i: (0, 0, i, 0, 0)),
            pl.BlockSpec((D, D), lambda i: (0, 0)),
            pl.BlockSpec((1, D), lambda i: (0, 0)),
        ],
        out_specs=pl.BlockSpec((_RB, D), lambda i: (i, 0)),
        out_shape=jax.ShapeDtypeStruct((N_NODES, D), jnp.float32),
    )(ph, pd5, wt, b2)


def kernel(inputs, edge_index, W, b):
    ei = edge_index.astype(jnp.int32)
    sidi = ei.reshape(2, NCHUNK, K)
    zh = jnp.zeros((RPS, D), jnp.float32)
    zd = jnp.zeros((DR, D), jnp.float32)
    ph, pd = _sc_aggregate(inputs, sidi, zh, zd)
    pd5 = pd.reshape(NC, NS, DR, 1, D)
    return _tc_linear(ph, pd5, W.T, b.reshape(1, D))
